# baseline (speedup 1.0000x reference)
import functools

import jax
import jax.numpy as jnp
from jax import lax
from jax.experimental import pallas as pl
from jax.experimental.pallas import tpu as pltpu
from jax.experimental.pallas import tpu_sc as plsc

N = 10000
E = 320000
D = 128

NC = 2
NS = 16
NW = NC * NS

GRP = 64
GPW = 160
SG = 16
BODY = 2 * SG
TSUP = GPW // BODY
NBUF = 4
EPAD = NW * GPW * GRP

NPAD = 10240
RPT = NPAD // NS


def _make_agg(compute_deg: bool):
  mesh = plsc.VectorSubcoreMesh(core_axis_name="c", subcore_axis_name="s",
                                num_cores=NC, num_subcores=NS)

  out_type = [jax.ShapeDtypeStruct((NPAD, D), jnp.float32),
              jax.ShapeDtypeStruct((NPAD, D), jnp.float32)]
  if compute_deg:
    out_type += [jax.ShapeDtypeStruct((NPAD,), jnp.float32),
                 jax.ShapeDtypeStruct((NPAD,), jnp.float32)]

  scratch = dict(
      ia_s=pltpu.VMEM((SG, GRP), jnp.int32),
      ia_d=pltpu.VMEM((SG, GRP), jnp.int32),
      ib_s=pltpu.VMEM((SG, GRP), jnp.int32),
      ib_d=pltpu.VMEM((SG, GRP), jnp.int32),
      rowsb=[pltpu.VMEM((GRP, D), jnp.float32) for _ in range(NBUF)],
      acc=pltpu.VMEM_SHARED((NPAD, D), jnp.float32),
      semb=[pltpu.SemaphoreType.DMA for _ in range(NBUF)],
      isema=pltpu.SemaphoreType.DMA,
      isemb=pltpu.SemaphoreType.DMA,
  )
  if compute_deg:
    scratch.update(
        ones=pltpu.VMEM((GRP,), jnp.float32),
        dacc=pltpu.VMEM_SHARED((NPAD,), jnp.float32),
    )

  def body(h_hbm, src_hbm, dst_hbm, zrows_hbm, zdeg_hbm,
           part0, part1, degp0, degp1,
           ia_s, ia_d, ib_s, ib_d, rowsb, acc, semb, isema, isemb,
           ones=None, dacc=None):
    c = lax.axis_index("c")
    s = lax.axis_index("s")
    w = s * NC + c
    r0 = s * RPT

    pltpu.sync_copy(zrows_hbm, acc.at[pl.ds(r0, RPT)])
    if compute_deg:
      pltpu.sync_copy(zdeg_hbm, dacc.at[pl.ds(r0, RPT)])
      for i in range(GRP // 16):
        ones[pl.ds(i * 16, 16)] = jnp.ones((16,), jnp.float32)
    plsc.subcore_barrier()


    def scat(idxd_set, loc, slot):
      pltpu.sync_copy(rowsb[slot], acc.at[idxd_set.at[loc]], add=True)
      if compute_deg:
        pltpu.sync_copy(ones, dacc.at[idxd_set.at[loc]], add=True)

    def load_set(set_s, set_d, base, sem):
      pltpu.async_copy(src_hbm.at[w, pl.ds(base, SG)], set_s, sem)
      pltpu.async_copy(dst_hbm.at[w, pl.ds(base, SG)], set_d, sem)

    def wait_set(set_s, set_d, base, sem):
      pltpu.make_async_copy(src_hbm.at[w, pl.ds(base, SG)], set_s, sem).wait()
      pltpu.make_async_copy(dst_hbm.at[w, pl.ds(base, SG)], set_d, sem).wait()

    def gissue(idx_set, loc, slot):
      pltpu.async_copy(h_hbm.at[idx_set.at[loc]], rowsb[slot], semb[slot])

    def gwait(idx_set, loc, slot):
      pltpu.make_async_copy(h_hbm.at[idx_set.at[loc]], rowsb[slot],
                            semb[slot]).wait()

    def body32(t, reload):
      for p in range(BODY):
        if p == SG - 3 and reload:
          wait_set(ib_s, ib_d, pl.multiple_of(BODY * t + SG, 8), isemb)
        if p == SG and reload:
          load_set(ia_s, ia_d, pl.multiple_of(BODY * t + BODY, 8), isema)
        if p == BODY - 3 and reload:
          wait_set(ia_s, ia_d, pl.multiple_of(BODY * t + BODY, 8), isema)
        tgt = p + NBUF - 1
        if tgt < BODY or reload:
          tloc = tgt % SG
          tset = ia_s if (tgt % BODY) < SG else ib_s
          gissue(tset, tloc, tgt % NBUF)
        if not reload and p == SG - 3:
          wait_set(ib_s, ib_d, BODY * t + SG, isemb)
        gwait(ia_s if p < SG else ib_s, p % SG, p % NBUF)
        scat(ia_d if p < SG else ib_d, p % SG, p % NBUF)
      if reload:
        load_set(ib_s, ib_d, pl.multiple_of(BODY * t + BODY + SG, 8), isemb)

    pltpu.sync_copy(src_hbm.at[w, pl.ds(0, SG)], ia_s)
    pltpu.sync_copy(dst_hbm.at[w, pl.ds(0, SG)], ia_d)
    load_set(ib_s, ib_d, SG, isemb)
    for b in range(NBUF - 1):
      gissue(ia_s, b, b)

    def super_body(t, carry):
      body32(t, True)
      return carry

    lax.fori_loop(0, TSUP - 1, super_body, 0)
    body32(TSUP - 1, False)

    plsc.subcore_barrier()

    @pl.when(c == 0)
    def _():
      pltpu.sync_copy(acc.at[pl.ds(r0, RPT)], part0.at[pl.ds(r0, RPT)])
      if compute_deg:
        pltpu.sync_copy(dacc.at[pl.ds(r0, RPT)], degp0.at[pl.ds(r0, RPT)])

    @pl.when(c == 1)
    def _():
      pltpu.sync_copy(acc.at[pl.ds(r0, RPT)], part1.at[pl.ds(r0, RPT)])
      if compute_deg:
        pltpu.sync_copy(dacc.at[pl.ds(r0, RPT)], degp1.at[pl.ds(r0, RPT)])

  if compute_deg:
    def wrapped(h, src, dst, zrows, zdeg, part0, part1, degp0, degp1,
                ia_s=None, ia_d=None, ib_s=None, ib_d=None, rowsb=None,
                acc=None, semb=None, isema=None, isemb=None,
                ones=None, dacc=None):
      body(h, src, dst, zrows, zdeg, part0, part1, degp0, degp1,
           ia_s, ia_d, ib_s, ib_d, rowsb, acc, semb, isema, isemb,
           ones, dacc)
  else:
    def wrapped(h, src, dst, zrows, part0, part1,
                ia_s=None, ia_d=None, ib_s=None, ib_d=None, rowsb=None,
                acc=None, semb=None, isema=None, isemb=None):
      body(h, src, dst, zrows, None, part0, part1, None, None,
           ia_s, ia_d, ib_s, ib_d, rowsb, acc, semb, isema, isemb)

  return pl.kernel(wrapped, out_type=tuple(out_type), mesh=mesh,
                   scratch_types=scratch)


_ROW_BLK = 5000


def _make_dense(relu: bool):
  def dense_body(p0_ref, p1_ref, d0_ref, d1_ref, h_ref, wl_ref, bl_ref,
                 wr_ref, o_ref):
    ssum = p0_ref[...] + p1_ref[...]
    d = d0_ref[...] + d1_ref[...]
    agg = ssum * (1.0 / jnp.maximum(d, 1.0))
    y = jnp.dot(agg, wl_ref[...], preferred_element_type=jnp.float32,
                precision=lax.Precision.HIGHEST)
    y = y + bl_ref[...]
    y = y + jnp.dot(h_ref[...], wr_ref[...], preferred_element_type=jnp.float32,
                    precision=lax.Precision.HIGHEST)
    o_ref[...] = jnp.maximum(y, 0.0) if relu else y

  return pl.pallas_call(
      dense_body,
      grid=(N // _ROW_BLK,),
      in_specs=[
          pl.BlockSpec((_ROW_BLK, D), lambda i: (i, 0)),
          pl.BlockSpec((_ROW_BLK, D), lambda i: (i, 0)),
          pl.BlockSpec((_ROW_BLK, 1), lambda i: (i, 0)),
          pl.BlockSpec((_ROW_BLK, 1), lambda i: (i, 0)),
          pl.BlockSpec((_ROW_BLK, D), lambda i: (i, 0)),
          pl.BlockSpec((D, D), lambda i: (0, 0)),
          pl.BlockSpec((1, D), lambda i: (0, 0)),
          pl.BlockSpec((D, D), lambda i: (0, 0)),
      ],
      out_specs=pl.BlockSpec((_ROW_BLK, D), lambda i: (i, 0)),
      out_shape=jax.ShapeDtypeStruct((N, D), jnp.float32),
  )


def kernel(x, edge_index, Wl1, bl1, Wr1, Wl2, bl2, Wr2, Wl3, bl3, Wr3):
  agg_with_deg = _make_agg(True)
  agg = _make_agg(False)
  dense_relu = _make_dense(True)
  dense_last = _make_dense(False)

  pad = EPAD - E
  pad_src = (jnp.arange(pad, dtype=jnp.int32) * 131) % N
  pad_dst = N + (jnp.arange(pad, dtype=jnp.int32) % (NPAD - N))
  src3 = jnp.concatenate(
      [edge_index[0].astype(jnp.int32), pad_src]).reshape(NW, GPW, GRP)
  dst3 = jnp.concatenate(
      [edge_index[1].astype(jnp.int32), pad_dst]).reshape(NW, GPW, GRP)
  zrows = jnp.zeros((RPT, D), jnp.float32)
  zdeg = jnp.zeros((RPT,), jnp.float32)

  p0, p1, dg0, dg1 = agg_with_deg(x, src3, dst3, zrows, zdeg)
  dg0 = dg0.reshape(NPAD, 1)
  dg1 = dg1.reshape(NPAD, 1)
  h1 = dense_relu(p0, p1, dg0, dg1, x, Wl1, bl1.reshape(1, D), Wr1)
  p0, p1 = agg(h1, src3, dst3, zrows)
  h2 = dense_relu(p0, p1, dg0, dg1, h1, Wl2, bl2.reshape(1, D), Wr2)
  p0, p1 = agg(h2, src3, dst3, zrows)
  return dense_last(p0, p1, dg0, dg1, h2, Wl3, bl3.reshape(1, D), Wr3)


# speedup vs baseline: 1.0869x; 1.0869x over previous
import functools

import jax
import jax.numpy as jnp
from jax import lax
from jax.experimental import pallas as pl
from jax.experimental.pallas import tpu as pltpu
from jax.experimental.pallas import tpu_sc as plsc

N = 10000
E = 320000
D = 128

NC = 2
NS = 16
NW = NC * NS

GRP = 64
GPW = 160
SG = 16
BODY = 2 * SG
TSUP = GPW // BODY
NBUF = 4
EPAD = NW * GPW * GRP

NPAD = 10240
RPT = NPAD // NS


def _make_agg(compute_deg: bool):
  mesh = plsc.VectorSubcoreMesh(core_axis_name="c", subcore_axis_name="s",
                                num_cores=NC, num_subcores=NS)

  out_type = [jax.ShapeDtypeStruct((NPAD, D), jnp.float32),
              jax.ShapeDtypeStruct((NPAD, D), jnp.float32)]
  if compute_deg:
    out_type += [jax.ShapeDtypeStruct((NPAD,), jnp.float32),
                 jax.ShapeDtypeStruct((NPAD,), jnp.float32)]

  scratch = dict(
      ia_s=pltpu.VMEM((SG, GRP), jnp.int32),
      ia_d=pltpu.VMEM((SG, GRP), jnp.int32),
      ib_s=pltpu.VMEM((SG, GRP), jnp.int32),
      ib_d=pltpu.VMEM((SG, GRP), jnp.int32),
      rowsb=[pltpu.VMEM((GRP, D), jnp.float32) for _ in range(NBUF)],
      acc=pltpu.VMEM_SHARED((NPAD, D), jnp.float32),
      semb=[pltpu.SemaphoreType.DMA for _ in range(NBUF)],
      isema=pltpu.SemaphoreType.DMA,
      isemb=pltpu.SemaphoreType.DMA,
      zsem=pltpu.SemaphoreType.DMA,
  )
  if compute_deg:
    scratch.update(
        ones=pltpu.VMEM((GRP,), jnp.float32),
        dacc=pltpu.VMEM_SHARED((NPAD,), jnp.float32),
    )

  def body(h_hbm, src_hbm, dst_hbm, zrows_hbm, zdeg_hbm,
           part0, part1, degp0, degp1,
           ia_s, ia_d, ib_s, ib_d, rowsb, acc, semb, isema, isemb, zsem,
           ones=None, dacc=None):
    c = lax.axis_index("c")
    s = lax.axis_index("s")
    w = s * NC + c
    r0 = s * RPT

    zcp = pltpu.async_copy(zrows_hbm, acc.at[pl.ds(r0, RPT)], zsem)
    zcpd = None
    if compute_deg:
      zcpd = pltpu.async_copy(zdeg_hbm, dacc.at[pl.ds(r0, RPT)], zsem)
      for i in range(GRP // 16):
        ones[pl.ds(i * 16, 16)] = jnp.ones((16,), jnp.float32)


    def scat(idxd_set, loc, slot):
      pltpu.sync_copy(rowsb[slot], acc.at[idxd_set.at[loc]], add=True)
      if compute_deg:
        pltpu.sync_copy(ones, dacc.at[idxd_set.at[loc]], add=True)

    def load_set(set_s, set_d, base, sem):
      pltpu.async_copy(src_hbm.at[w, pl.ds(base, SG)], set_s, sem)
      pltpu.async_copy(dst_hbm.at[w, pl.ds(base, SG)], set_d, sem)

    def wait_set(set_s, set_d, base, sem):
      pltpu.make_async_copy(src_hbm.at[w, pl.ds(base, SG)], set_s, sem).wait()
      pltpu.make_async_copy(dst_hbm.at[w, pl.ds(base, SG)], set_d, sem).wait()

    def gissue(idx_set, loc, slot):
      pltpu.async_copy(h_hbm.at[idx_set.at[loc]], rowsb[slot], semb[slot])

    def gwait(idx_set, loc, slot):
      pltpu.make_async_copy(h_hbm.at[idx_set.at[loc]], rowsb[slot],
                            semb[slot]).wait()

    def body32(t, reload):
      for p in range(BODY):
        if p == SG - 3 and reload:
          wait_set(ib_s, ib_d, pl.multiple_of(BODY * t + SG, 8), isemb)
        if p == SG and reload:
          load_set(ia_s, ia_d, pl.multiple_of(BODY * t + BODY, 8), isema)
        if p == BODY - 3 and reload:
          wait_set(ia_s, ia_d, pl.multiple_of(BODY * t + BODY, 8), isema)
        tgt = p + NBUF - 1
        if tgt < BODY or reload:
          tloc = tgt % SG
          tset = ia_s if (tgt % BODY) < SG else ib_s
          gissue(tset, tloc, tgt % NBUF)
        if not reload and p == SG - 3:
          wait_set(ib_s, ib_d, BODY * t + SG, isemb)
        gwait(ia_s if p < SG else ib_s, p % SG, p % NBUF)
        scat(ia_d if p < SG else ib_d, p % SG, p % NBUF)
      if reload:
        load_set(ib_s, ib_d, pl.multiple_of(BODY * t + BODY + SG, 8), isemb)

    pltpu.sync_copy(src_hbm.at[w, pl.ds(0, SG)], ia_s)
    pltpu.sync_copy(dst_hbm.at[w, pl.ds(0, SG)], ia_d)
    load_set(ib_s, ib_d, SG, isemb)
    for b in range(NBUF - 1):
      gissue(ia_s, b, b)

    zcp.wait()
    if compute_deg:
      zcpd.wait()
    plsc.subcore_barrier()

    def super_body(t, carry):
      body32(t, True)
      return carry

    lax.fori_loop(0, TSUP - 1, super_body, 0)
    body32(TSUP - 1, False)

    plsc.subcore_barrier()

    @pl.when(c == 0)
    def _():
      pltpu.sync_copy(acc.at[pl.ds(r0, RPT)], part0.at[pl.ds(r0, RPT)])
      if compute_deg:
        pltpu.sync_copy(dacc.at[pl.ds(r0, RPT)], degp0.at[pl.ds(r0, RPT)])

    @pl.when(c == 1)
    def _():
      pltpu.sync_copy(acc.at[pl.ds(r0, RPT)], part1.at[pl.ds(r0, RPT)])
      if compute_deg:
        pltpu.sync_copy(dacc.at[pl.ds(r0, RPT)], degp1.at[pl.ds(r0, RPT)])

  if compute_deg:
    def wrapped(h, src, dst, zrows, zdeg, part0, part1, degp0, degp1,
                ia_s=None, ia_d=None, ib_s=None, ib_d=None, rowsb=None,
                acc=None, semb=None, isema=None, isemb=None, zsem=None,
                ones=None, dacc=None):
      body(h, src, dst, zrows, zdeg, part0, part1, degp0, degp1,
           ia_s, ia_d, ib_s, ib_d, rowsb, acc, semb, isema, isemb, zsem,
           ones, dacc)
  else:
    def wrapped(h, src, dst, zrows, part0, part1,
                ia_s=None, ia_d=None, ib_s=None, ib_d=None, rowsb=None,
                acc=None, semb=None, isema=None, isemb=None, zsem=None):
      body(h, src, dst, zrows, None, part0, part1, None, None,
           ia_s, ia_d, ib_s, ib_d, rowsb, acc, semb, isema, isemb, zsem)

  return pl.kernel(wrapped, out_type=tuple(out_type), mesh=mesh,
                   scratch_types=scratch)


_ROW_BLK = 2000


def _make_dense(relu: bool):
  def dense_body(p0_ref, p1_ref, d0_ref, d1_ref, h_ref, wl_ref, bl_ref,
                 wr_ref, o_ref):
    ssum = p0_ref[...] + p1_ref[...]
    d = d0_ref[...] + d1_ref[...]
    agg = ssum * (1.0 / jnp.maximum(d, 1.0))
    y = jnp.dot(agg, wl_ref[...], preferred_element_type=jnp.float32,
                precision=lax.Precision.HIGHEST)
    y = y + bl_ref[...]
    y = y + jnp.dot(h_ref[...], wr_ref[...], preferred_element_type=jnp.float32,
                    precision=lax.Precision.HIGHEST)
    o_ref[...] = jnp.maximum(y, 0.0) if relu else y

  return pl.pallas_call(
      dense_body,
      grid=(N // _ROW_BLK,),
      in_specs=[
          pl.BlockSpec((_ROW_BLK, D), lambda i: (i, 0)),
          pl.BlockSpec((_ROW_BLK, D), lambda i: (i, 0)),
          pl.BlockSpec((_ROW_BLK, 1), lambda i: (i, 0)),
          pl.BlockSpec((_ROW_BLK, 1), lambda i: (i, 0)),
          pl.BlockSpec((_ROW_BLK, D), lambda i: (i, 0)),
          pl.BlockSpec((D, D), lambda i: (0, 0)),
          pl.BlockSpec((1, D), lambda i: (0, 0)),
          pl.BlockSpec((D, D), lambda i: (0, 0)),
      ],
      out_specs=pl.BlockSpec((_ROW_BLK, D), lambda i: (i, 0)),
      out_shape=jax.ShapeDtypeStruct((N, D), jnp.float32),
  )


def kernel(x, edge_index, Wl1, bl1, Wr1, Wl2, bl2, Wr2, Wl3, bl3, Wr3):
  agg_with_deg = _make_agg(True)
  agg = _make_agg(False)
  dense_relu = _make_dense(True)
  dense_last = _make_dense(False)

  pad = EPAD - E
  pad_src = (jnp.arange(pad, dtype=jnp.int32) * 131) % N
  pad_dst = N + (jnp.arange(pad, dtype=jnp.int32) % (NPAD - N))
  src3 = jnp.concatenate(
      [edge_index[0].astype(jnp.int32), pad_src]).reshape(NW, GPW, GRP)
  dst3 = jnp.concatenate(
      [edge_index[1].astype(jnp.int32), pad_dst]).reshape(NW, GPW, GRP)
  zrows = jnp.zeros((RPT, D), jnp.float32)
  zdeg = jnp.zeros((RPT,), jnp.float32)

  p0, p1, dg0, dg1 = agg_with_deg(x, src3, dst3, zrows, zdeg)
  dg0 = dg0.reshape(NPAD, 1)
  dg1 = dg1.reshape(NPAD, 1)
  h1 = dense_relu(p0, p1, dg0, dg1, x, Wl1, bl1.reshape(1, D), Wr1)
  p0, p1 = agg(h1, src3, dst3, zrows)
  h2 = dense_relu(p0, p1, dg0, dg1, h1, Wl2, bl2.reshape(1, D), Wr2)
  p0, p1 = agg(h2, src3, dst3, zrows)
  return dense_last(p0, p1, dg0, dg1, h2, Wl3, bl3.reshape(1, D), Wr3)
